# baseline (speedup 1.0000x reference)
import jax
import jax.numpy as jnp
from jax import lax
from jax.experimental import pallas as pl
from jax.experimental.pallas import tpu as pltpu
from jax.experimental.pallas import tpu_sc as plsc

N = 10000
K = 32
D = 128
DH = D // 2

_INFO = plsc.get_sparse_core_info()
NC = _INFO.num_cores
NS = _INFO.num_subcores
NW = NC * NS
P0 = 224
P1 = 416
CUT = NS * P0
NPAD = NS * (P0 + P1)
NBUF = 2
BATCH = 4
RPB = K * BATCH
LANES = 16
MASK_HI = -65536


def _stats_body(f_ref, sum_ref, max_ref, pack_ref):
    f = f_ref[:]
    sum_ref[:] = jnp.sum(f, axis=1, keepdims=True)
    max_ref[:] = jnp.max(f).reshape(1, 1)
    lo = lax.bitcast_convert_type(
        f[:, :DH].astype(jnp.bfloat16), jnp.uint16).astype(jnp.uint32)
    hi = lax.bitcast_convert_type(
        f[:, DH:].astype(jnp.bfloat16), jnp.uint16).astype(jnp.uint32)
    pack_ref[:] = lax.bitcast_convert_type(
        lo | (hi << jnp.uint32(16)), jnp.int32)


def _row_stats(features):
    return pl.pallas_call(
        _stats_body,
        out_shape=(
            jax.ShapeDtypeStruct((N, 1), jnp.float32),
            jax.ShapeDtypeStruct((1, 1), jnp.float32),
            jax.ShapeDtypeStruct((N, DH), jnp.int32),
        ),
    )(features)


def _sc_body(packed_hbm, nbrs_hbm, rowsum_hbm,
             acc_hbm, cnt_hbm,
             nbrs_v, rowsum_v, rows_v, acc_v, cnt_v, *sems):
    cid = lax.axis_index("c")
    sid = lax.axis_index("s")
    base = jnp.where(cid == 0, sid * P0, CUT + sid * P1)
    nbatch = jnp.where(cid == 0, P0 // BATCH, P1 // BATCH)

    @pl.when(cid == 0)
    def _stage_n0():
        pltpu.sync_copy(nbrs_hbm.at[pl.ds(base * K, P0 * K)],
                        nbrs_v.at[pl.ds(0, P0 * K)])

    @pl.when(cid != 0)
    def _stage_n1():
        pltpu.sync_copy(nbrs_hbm.at[pl.ds(base * K, P1 * K)], nbrs_v)
    pltpu.sync_copy(rowsum_hbm, rowsum_v)

    lanes = lax.iota(jnp.int32, LANES)

    def count_group(g, _):
        flatbase = (g * LANES + lanes) * K
        cnt = jnp.zeros((LANES,), jnp.float32)
        for k in range(K):
            ids = plsc.load_gather(nbrs_v, [flatbase + k])
            vals = plsc.load_gather(rowsum_v, [ids])
            cnt = cnt + jnp.where(vals != 0.0, 1.0, 0.0)
        cnt_v[pl.ds(g * LANES, LANES)] = cnt
        return _

    lax.fori_loop(0, jnp.where(cid == 0, P0 // LANES, P1 // LANES),
                  count_group, None)

    def idx_at(j):
        return nbrs_v.at[pl.ds(j * RPB, RPB)]

    for b in range(NBUF):
        pltpu.async_copy(packed_hbm.at[idx_at(b)], rows_v.at[b], sems[b])

    waiters = [
        pltpu.make_async_copy(packed_hbm.at[idx_at(b)], rows_v.at[b],
                              sems[b])
        for b in range(NBUF)
    ]

    def ring_step(it, _):
        j0 = it * NBUF
        for b in range(NBUF):
            j = j0 + b
            waiters[b].wait()
            for nb in range(BATCH):
                row0 = nb * K
                node = j * BATCH + nb
                for c16 in range(DH // LANES):
                    sl = pl.ds(c16 * LANES, LANES)
                    t = rows_v[b, row0, sl]
                    alo = plsc.bitcast(lax.shift_left(t, 16), jnp.float32)
                    ahi = plsc.bitcast(t & MASK_HI, jnp.float32)
                    for k in range(1, K):
                        t = rows_v[b, row0 + k, sl]
                        alo = alo + plsc.bitcast(
                            lax.shift_left(t, 16), jnp.float32)
                        ahi = ahi + plsc.bitcast(t & MASK_HI, jnp.float32)
                    acc_v[node, pl.ds(c16 * LANES, LANES)] = alo
                    acc_v[node, pl.ds(DH + c16 * LANES, LANES)] = ahi
            nxt = j + NBUF

            @pl.when(nxt < nbatch)
            def _fire():
                pltpu.async_copy(
                    packed_hbm.at[idx_at(nxt)], rows_v.at[b], sems[b])
        return _

    lax.fori_loop(0, nbatch // NBUF, ring_step, None)

    @pl.when(cid == 0)
    def _out0():
        pltpu.sync_copy(acc_v.at[pl.ds(0, P0)], acc_hbm.at[pl.ds(base, P0)])
        pltpu.sync_copy(cnt_v.at[pl.ds(0, P0)], cnt_hbm.at[pl.ds(base, P0)])

    @pl.when(cid != 0)
    def _out1():
        pltpu.sync_copy(acc_v, acc_hbm.at[pl.ds(base, P1)])
        pltpu.sync_copy(cnt_v, cnt_hbm.at[pl.ds(base, P1)])


def _sc_gather(packed, nbrs_pad, rowsum):
    mesh = plsc.VectorSubcoreMesh(core_axis_name="c", subcore_axis_name="s")
    run = pl.kernel(
        _sc_body,
        out_type=(
            jax.ShapeDtypeStruct((NPAD, D), jnp.float32),
            jax.ShapeDtypeStruct((NPAD,), jnp.float32),
        ),
        mesh=mesh,
        scratch_types=[
            pltpu.VMEM((P1 * K,), jnp.int32),
            pltpu.VMEM((N,), jnp.float32),
            pltpu.VMEM((NBUF, RPB, DH), jnp.int32),
            pltpu.VMEM((P1, D), jnp.float32),
            pltpu.VMEM((P1,), jnp.float32),
        ] + [pltpu.SemaphoreType.DMA] * NBUF,
        compiler_params=pltpu.CompilerParams(
            needs_layout_passes=False, use_tc_tiling_on_sc=False),
    )
    return run(packed, nbrs_pad, rowsum)


def _finish_body(f_ref, a_ref, c_ref, m_ref, o_ref):
    m = m_ref[0, 0] + 1e-6
    f = f_ref[:] / m
    num = jnp.maximum(c_ref[:], 1.0)
    mean = a_ref[:] / m / num
    x = f - mean
    local = jnp.maximum(x, 0.0) + jnp.log1p(jnp.exp(-jnp.abs(x)))
    dmax = jnp.max(f, axis=1, keepdims=True)
    o_ref[:] = jnp.max(local * (f / (1e-6 + dmax)), axis=1, keepdims=True)


def _finish(features, acc, cnt, mx):
    return pl.pallas_call(
        _finish_body,
        out_shape=jax.ShapeDtypeStruct((N, 1), jnp.float32),
    )(features, acc, cnt, mx)


@jax.jit
def _run(features, neighbors):
    rowsum2d, mx, packed = _row_stats(features)
    rowsum = rowsum2d.reshape(N)
    nbrs_pad = jnp.pad(neighbors, ((0, NPAD - N), (0, 0))).reshape(NPAD * K)
    acc, cnt = _sc_gather(packed, nbrs_pad, rowsum)
    return _finish(features, acc[:N], cnt[:N].reshape(N, 1), mx)


def kernel(features, neighbors, first_pcd_length, second_pcd_length):
    return _run(features, neighbors)


# speedup vs baseline: 1.1895x; 1.1895x over previous
import jax
import jax.numpy as jnp
from jax import lax
from jax.experimental import pallas as pl
from jax.experimental.pallas import tpu as pltpu
from jax.experimental.pallas import tpu_sc as plsc

N = 10000
K = 32
D = 128
DH = D // 2

_INFO = plsc.get_sparse_core_info()
NC = _INFO.num_cores
NS = _INFO.num_subcores
NW = NC * NS
P0 = 416
P1 = 224
CUT = NS * P0
NPAD = NS * (P0 + P1)
NBUF = 2
BATCH = 4
RPB = K * BATCH
LANES = 16
MASK_HI = -65536


def _stats_body(f_ref, sum_ref, max_ref, pack_ref):
    f = f_ref[:]
    sum_ref[:] = jnp.sum(f, axis=1, keepdims=True)
    max_ref[:] = jnp.max(f).reshape(1, 1)
    lo = lax.bitcast_convert_type(
        f[:, :DH].astype(jnp.bfloat16), jnp.uint16).astype(jnp.uint32)
    hi = lax.bitcast_convert_type(
        f[:, DH:].astype(jnp.bfloat16), jnp.uint16).astype(jnp.uint32)
    pack_ref[:] = lax.bitcast_convert_type(
        lo | (hi << jnp.uint32(16)), jnp.int32)


def _row_stats(features):
    return pl.pallas_call(
        _stats_body,
        out_shape=(
            jax.ShapeDtypeStruct((N, 1), jnp.float32),
            jax.ShapeDtypeStruct((1, 1), jnp.float32),
            jax.ShapeDtypeStruct((N, DH), jnp.int32),
        ),
    )(features)


def _sc_body(packed_hbm, nbrs_hbm, rowsum_hbm,
             acc_hbm, cnt_hbm,
             nbrs_v, rowsum_v, rows_v, acc_v, cnt_v, *sems):
    cid = lax.axis_index("c")
    sid = lax.axis_index("s")
    base = jnp.where(cid == 0, sid * P0, CUT + sid * P1)
    nbatch = jnp.where(cid == 0, P0 // BATCH, P1 // BATCH)

    @pl.when(cid == 0)
    def _stage_n0():
        pltpu.sync_copy(nbrs_hbm.at[pl.ds(base * K, P0 * K)],
                        nbrs_v.at[pl.ds(0, P0 * K)])

    @pl.when(cid != 0)
    def _stage_n1():
        pltpu.sync_copy(nbrs_hbm.at[pl.ds(base * K, P1 * K)],
                        nbrs_v.at[pl.ds(0, P1 * K)])
    pltpu.sync_copy(rowsum_hbm, rowsum_v)

    lanes = lax.iota(jnp.int32, LANES)

    def count_group(g, _):
        flatbase = (g * LANES + lanes) * K
        cnt = jnp.zeros((LANES,), jnp.float32)
        for k in range(K):
            ids = plsc.load_gather(nbrs_v, [flatbase + k])
            vals = plsc.load_gather(rowsum_v, [ids])
            cnt = cnt + jnp.where(vals != 0.0, 1.0, 0.0)
        cnt_v[pl.ds(g * LANES, LANES)] = cnt
        return _

    lax.fori_loop(0, jnp.where(cid == 0, P0 // LANES, P1 // LANES),
                  count_group, None)

    def idx_at(j):
        return nbrs_v.at[pl.ds(j * RPB, RPB)]

    for b in range(NBUF):
        pltpu.async_copy(packed_hbm.at[idx_at(b)], rows_v.at[b], sems[b])

    waiters = [
        pltpu.make_async_copy(packed_hbm.at[idx_at(b)], rows_v.at[b],
                              sems[b])
        for b in range(NBUF)
    ]

    def ring_step(it, _):
        j0 = it * NBUF
        for b in range(NBUF):
            j = j0 + b
            waiters[b].wait()
            for nb in range(BATCH):
                row0 = nb * K
                node = j * BATCH + nb
                for c16 in range(DH // LANES):
                    sl = pl.ds(c16 * LANES, LANES)
                    t = rows_v[b, row0, sl]
                    alo = plsc.bitcast(lax.shift_left(t, 16), jnp.float32)
                    ahi = plsc.bitcast(t & MASK_HI, jnp.float32)
                    for k in range(1, K):
                        t = rows_v[b, row0 + k, sl]
                        alo = alo + plsc.bitcast(
                            lax.shift_left(t, 16), jnp.float32)
                        ahi = ahi + plsc.bitcast(t & MASK_HI, jnp.float32)
                    acc_v[node, pl.ds(c16 * LANES, LANES)] = alo
                    acc_v[node, pl.ds(DH + c16 * LANES, LANES)] = ahi
            nxt = j + NBUF

            @pl.when(nxt < nbatch)
            def _fire():
                pltpu.async_copy(
                    packed_hbm.at[idx_at(nxt)], rows_v.at[b], sems[b])
        return _

    lax.fori_loop(0, nbatch // NBUF, ring_step, None)

    @pl.when(cid == 0)
    def _out0():
        pltpu.sync_copy(acc_v.at[pl.ds(0, P0)], acc_hbm.at[pl.ds(base, P0)])
        pltpu.sync_copy(cnt_v.at[pl.ds(0, P0)], cnt_hbm.at[pl.ds(base, P0)])

    @pl.when(cid != 0)
    def _out1():
        pltpu.sync_copy(acc_v.at[pl.ds(0, P1)], acc_hbm.at[pl.ds(base, P1)])
        pltpu.sync_copy(cnt_v.at[pl.ds(0, P1)], cnt_hbm.at[pl.ds(base, P1)])


def _sc_gather(packed, nbrs_pad, rowsum):
    mesh = plsc.VectorSubcoreMesh(core_axis_name="c", subcore_axis_name="s")
    run = pl.kernel(
        _sc_body,
        out_type=(
            jax.ShapeDtypeStruct((NPAD, D), jnp.float32),
            jax.ShapeDtypeStruct((NPAD,), jnp.float32),
        ),
        mesh=mesh,
        scratch_types=[
            pltpu.VMEM((P0 * K,), jnp.int32),
            pltpu.VMEM((N,), jnp.float32),
            pltpu.VMEM((NBUF, RPB, DH), jnp.int32),
            pltpu.VMEM((P0, D), jnp.float32),
            pltpu.VMEM((P0,), jnp.float32),
        ] + [pltpu.SemaphoreType.DMA] * NBUF,
        compiler_params=pltpu.CompilerParams(
            needs_layout_passes=False, use_tc_tiling_on_sc=False),
    )
    return run(packed, nbrs_pad, rowsum)


def _finish_body(f_ref, a_ref, c_ref, m_ref, o_ref):
    m = m_ref[0, 0] + 1e-6
    f = f_ref[:] / m
    num = jnp.maximum(c_ref[:], 1.0)
    mean = a_ref[:] / m / num
    x = f - mean
    local = jnp.maximum(x, 0.0) + jnp.log1p(jnp.exp(-jnp.abs(x)))
    dmax = jnp.max(f, axis=1, keepdims=True)
    o_ref[:] = jnp.max(local * (f / (1e-6 + dmax)), axis=1, keepdims=True)


def _finish(features, acc, cnt, mx):
    return pl.pallas_call(
        _finish_body,
        out_shape=jax.ShapeDtypeStruct((N, 1), jnp.float32),
    )(features, acc, cnt, mx)


@jax.jit
def _run(features, neighbors):
    rowsum2d, mx, packed = _row_stats(features)
    rowsum = rowsum2d.reshape(N)
    nbrs_pad = jnp.pad(neighbors, ((0, NPAD - N), (0, 0))).reshape(NPAD * K)
    acc, cnt = _sc_gather(packed, nbrs_pad, rowsum)
    return _finish(features, acc[:N], cnt[:N].reshape(N, 1), mx)


def kernel(features, neighbors, first_pcd_length, second_pcd_length):
    return _run(features, neighbors)


# speedup vs baseline: 1.3125x; 1.1034x over previous
import jax
import jax.numpy as jnp
from jax import lax
from jax.experimental import pallas as pl
from jax.experimental.pallas import tpu as pltpu
from jax.experimental.pallas import tpu_sc as plsc

N = 10000
K = 32
D = 128
DH = D // 2

_INFO = plsc.get_sparse_core_info()
NC = _INFO.num_cores
NS = _INFO.num_subcores
NW = NC * NS
P0 = 384
P1 = 256
CUT = NS * P0
NPAD = NS * (P0 + P1)
NBUF = 2
BATCH = 4
RPB = K * BATCH
LANES = 16
MASK_HI = -65536


def _stats_body(f_ref, sum_ref, max_ref, pack_ref):
    f = f_ref[:]
    sum_ref[:] = jnp.sum(f, axis=1, keepdims=True)
    max_ref[:] = jnp.max(f).reshape(1, 1)
    lo = lax.bitcast_convert_type(
        f[:, :DH].astype(jnp.bfloat16), jnp.uint16).astype(jnp.uint32)
    hi = lax.bitcast_convert_type(
        f[:, DH:].astype(jnp.bfloat16), jnp.uint16).astype(jnp.uint32)
    pack_ref[:] = lax.bitcast_convert_type(
        lo | (hi << jnp.uint32(16)), jnp.int32)


def _row_stats(features):
    return pl.pallas_call(
        _stats_body,
        out_shape=(
            jax.ShapeDtypeStruct((N, 1), jnp.float32),
            jax.ShapeDtypeStruct((1, 1), jnp.float32),
            jax.ShapeDtypeStruct((N, DH), jnp.int32),
        ),
    )(features)


def _sc_body(packed_hbm, nbrs_hbm, rowsum_hbm,
             acc_hbm, cnt_hbm,
             nbrs_v, rowsum_v, rows_v, acc_v, cnt_v, *sems):
    cid = lax.axis_index("c")
    sid = lax.axis_index("s")
    base = jnp.where(cid == 0, sid * P0, CUT + sid * P1)
    nbatch = jnp.where(cid == 0, P0 // BATCH, P1 // BATCH)

    @pl.when(cid == 0)
    def _stage_n0():
        pltpu.sync_copy(nbrs_hbm.at[pl.ds(base * K, P0 * K)],
                        nbrs_v.at[pl.ds(0, P0 * K)])

    @pl.when(cid != 0)
    def _stage_n1():
        pltpu.sync_copy(nbrs_hbm.at[pl.ds(base * K, P1 * K)],
                        nbrs_v.at[pl.ds(0, P1 * K)])
    pltpu.sync_copy(rowsum_hbm, rowsum_v)

    lanes = lax.iota(jnp.int32, LANES)

    def count_group(g, _):
        flatbase = (g * LANES + lanes) * K
        cnt = jnp.zeros((LANES,), jnp.float32)
        for k in range(K):
            ids = plsc.load_gather(nbrs_v, [flatbase + k])
            vals = plsc.load_gather(rowsum_v, [ids])
            cnt = cnt + jnp.where(vals != 0.0, 1.0, 0.0)
        cnt_v[pl.ds(g * LANES, LANES)] = cnt
        return _

    lax.fori_loop(0, jnp.where(cid == 0, P0 // LANES, P1 // LANES),
                  count_group, None)

    def idx_at(j):
        return nbrs_v.at[pl.ds(j * RPB, RPB)]

    for b in range(NBUF):
        pltpu.async_copy(packed_hbm.at[idx_at(b)], rows_v.at[b], sems[b])

    waiters = [
        pltpu.make_async_copy(packed_hbm.at[idx_at(b)], rows_v.at[b],
                              sems[b])
        for b in range(NBUF)
    ]

    def ring_step(it, _):
        j0 = it * NBUF
        for b in range(NBUF):
            j = j0 + b
            waiters[b].wait()
            for nb in range(BATCH):
                row0 = nb * K
                node = j * BATCH + nb
                for c16 in range(DH // LANES):
                    sl = pl.ds(c16 * LANES, LANES)
                    t = rows_v[b, row0, sl]
                    alo = plsc.bitcast(lax.shift_left(t, 16), jnp.float32)
                    ahi = plsc.bitcast(t & MASK_HI, jnp.float32)
                    for k in range(1, K):
                        t = rows_v[b, row0 + k, sl]
                        alo = alo + plsc.bitcast(
                            lax.shift_left(t, 16), jnp.float32)
                        ahi = ahi + plsc.bitcast(t & MASK_HI, jnp.float32)
                    acc_v[node, pl.ds(c16 * LANES, LANES)] = alo
                    acc_v[node, pl.ds(DH + c16 * LANES, LANES)] = ahi
            nxt = j + NBUF

            @pl.when(nxt < nbatch)
            def _fire():
                pltpu.async_copy(
                    packed_hbm.at[idx_at(nxt)], rows_v.at[b], sems[b])
        return _

    lax.fori_loop(0, nbatch // NBUF, ring_step, None)

    @pl.when(cid == 0)
    def _out0():
        pltpu.sync_copy(acc_v.at[pl.ds(0, P0)], acc_hbm.at[pl.ds(base, P0)])
        pltpu.sync_copy(cnt_v.at[pl.ds(0, P0)], cnt_hbm.at[pl.ds(base, P0)])

    @pl.when(cid != 0)
    def _out1():
        pltpu.sync_copy(acc_v.at[pl.ds(0, P1)], acc_hbm.at[pl.ds(base, P1)])
        pltpu.sync_copy(cnt_v.at[pl.ds(0, P1)], cnt_hbm.at[pl.ds(base, P1)])


def _sc_gather(packed, nbrs_pad, rowsum):
    mesh = plsc.VectorSubcoreMesh(core_axis_name="c", subcore_axis_name="s")
    run = pl.kernel(
        _sc_body,
        out_type=(
            jax.ShapeDtypeStruct((NPAD, D), jnp.float32),
            jax.ShapeDtypeStruct((NPAD,), jnp.float32),
        ),
        mesh=mesh,
        scratch_types=[
            pltpu.VMEM((P0 * K,), jnp.int32),
            pltpu.VMEM((N,), jnp.float32),
            pltpu.VMEM((NBUF, RPB, DH), jnp.int32),
            pltpu.VMEM((P0, D), jnp.float32),
            pltpu.VMEM((P0,), jnp.float32),
        ] + [pltpu.SemaphoreType.DMA] * NBUF,
        compiler_params=pltpu.CompilerParams(
            needs_layout_passes=False, use_tc_tiling_on_sc=False),
    )
    return run(packed, nbrs_pad, rowsum)


def _finish_body(f_ref, a_ref, c_ref, m_ref, o_ref):
    m = m_ref[0, 0] + 1e-6
    f = f_ref[:] / m
    num = jnp.maximum(c_ref[pl.ds(0, N), :], 1.0)
    mean = a_ref[pl.ds(0, N), :] / m / num
    x = f - mean
    local = jnp.maximum(x, 0.0) + jnp.log1p(jnp.exp(-jnp.abs(x)))
    dmax = jnp.max(f, axis=1, keepdims=True)
    o_ref[:] = jnp.max(local * (f / (1e-6 + dmax)), axis=1, keepdims=True)


def _finish(features, acc, cnt, mx):
    return pl.pallas_call(
        _finish_body,
        out_shape=jax.ShapeDtypeStruct((N, 1), jnp.float32),
    )(features, acc, cnt, mx)


@jax.jit
def _run(features, neighbors):
    rowsum2d, mx, packed = _row_stats(features)
    rowsum = rowsum2d.reshape(N)
    nbrs_pad = jnp.pad(neighbors, ((0, NPAD - N), (0, 0))).reshape(NPAD * K)
    acc, cnt = _sc_gather(packed, nbrs_pad, rowsum)
    return _finish(features, acc, cnt.reshape(NPAD, 1), mx)


def kernel(features, neighbors, first_pcd_length, second_pcd_length):
    return _run(features, neighbors)
